# baseline (speedup 1.0000x reference)
import functools

import jax
import jax.numpy as jnp
from jax import lax
from jax.experimental import pallas as pl
from jax.experimental.pallas import tpu as pltpu
from jax.experimental.pallas import tpu_sc as plsc

_NUM_WORKERS = 32
_CA = 16
_CB = 8


def _gather_rows(ids_flat, table):
    n = ids_flat.shape[0]
    v_rows, d = table.shape
    rows_per_worker = n // _NUM_WORKERS
    n_rounds = rows_per_worker // (_CA + _CB)
    assert n_rounds * (_CA + _CB) + _CA == rows_per_worker

    mesh = plsc.VectorSubcoreMesh(core_axis_name="c", subcore_axis_name="s")
    num_cores = mesh.num_cores

    @functools.partial(
        pl.kernel,
        out_type=jax.ShapeDtypeStruct((n, d), jnp.float32),
        mesh=mesh,
        scratch_types=[
            pltpu.VMEM((rows_per_worker,), jnp.int32),
            pltpu.VMEM((_CA, d), jnp.float32),
            pltpu.VMEM((_CB, d), jnp.float32),
            pltpu.SemaphoreType.DMA,
            pltpu.SemaphoreType.DMA,
            pltpu.SemaphoreType.DMA,
            pltpu.SemaphoreType.DMA,
        ],
    )
    def body(ids_hbm, table_hbm, out_hbm, idx_v, buf_a, buf_b, ga, sa, gb, sb):
        wid = lax.axis_index("s") * num_cores + lax.axis_index("c")
        base = wid * rows_per_worker
        pltpu.sync_copy(ids_hbm.at[pl.ds(base, rows_per_worker)], idx_v)

        def row_a(r):
            return r * (_CA + _CB)

        def row_b(r):
            return r * (_CA + _CB) + _CA

        def start_gather(row, buf, sem, l_i):
            off = pl.multiple_of(row, 8)
            pltpu.async_copy(
                table_hbm.at[idx_v.at[pl.ds(off, l_i)]], buf, sem)

        def start_store(row, buf, sem, l_i):
            pltpu.async_copy(buf, out_hbm.at[pl.ds(base + row, l_i)], sem)

        def wait_gather(buf, sem, l_i):
            pltpu.make_async_copy(
                table_hbm.at[idx_v.at[pl.ds(0, l_i)]], buf, sem).wait()

        def wait_store(buf, sem, l_i):
            pltpu.make_async_copy(
                buf, out_hbm.at[pl.ds(base, l_i)], sem).wait()

        start_gather(row_a(0), buf_a, ga, _CA)
        start_gather(row_b(0), buf_b, gb, _CB)

        for r in range(n_rounds + 1):
            wait_gather(buf_a, ga, _CA)
            start_store(row_a(r), buf_a, sa, _CA)
            if r < n_rounds:
                wait_gather(buf_b, gb, _CB)
                start_store(row_b(r), buf_b, sb, _CB)
            if r + 1 <= n_rounds:
                wait_store(buf_a, sa, _CA)
                start_gather(row_a(r + 1), buf_a, ga, _CA)
            if r + 1 <= n_rounds - 1:
                wait_store(buf_b, sb, _CB)
                start_gather(row_b(r + 1), buf_b, gb, _CB)

        wait_store(buf_a, sa, _CA)
        wait_store(buf_b, sb, _CB)

    return body(ids_flat, table)


def kernel(input_ids, table):
    b, s = input_ids.shape
    d = table.shape[1]
    out = _gather_rows(input_ids.reshape(b * s), table)
    return out.reshape(b, s, d)


# speedup vs baseline: 1.0838x; 1.0838x over previous
import functools

import jax
import jax.numpy as jnp
from jax import lax
from jax.experimental import pallas as pl
from jax.experimental.pallas import tpu as pltpu
from jax.experimental.pallas import tpu_sc as plsc

_NUM_WORKERS = 32
_CHUNK = 8


def _gather_rows(ids_flat, table):
    n = ids_flat.shape[0]
    v_rows, d = table.shape
    rows_per_worker = n // _NUM_WORKERS
    n_chunks = rows_per_worker // _CHUNK
    n_main = (n_chunks - 2) // 3 * 3
    assert n_chunks - n_main == 2

    mesh = plsc.VectorSubcoreMesh(core_axis_name="c", subcore_axis_name="s")
    num_cores = mesh.num_cores

    @functools.partial(
        pl.kernel,
        out_type=jax.ShapeDtypeStruct((n, d), jnp.float32),
        mesh=mesh,
        scratch_types=[
            pltpu.VMEM((rows_per_worker,), jnp.int32),
            pltpu.VMEM((3, _CHUNK, d), jnp.float32),
            pltpu.SemaphoreType.DMA,
            pltpu.SemaphoreType.DMA,
            pltpu.SemaphoreType.DMA,
            pltpu.SemaphoreType.DMA,
            pltpu.SemaphoreType.DMA,
            pltpu.SemaphoreType.DMA,
        ],
    )
    def body(ids_hbm, table_hbm, out_hbm, idx_v, bufs, g0, g1, g2, s0, s1, s2):
        gsem = (g0, g1, g2)
        ssem = (s0, s1, s2)
        wid = lax.axis_index("s") * num_cores + lax.axis_index("c")
        base = wid * rows_per_worker
        pltpu.sync_copy(ids_hbm.at[pl.ds(base, rows_per_worker)], idx_v)

        def start_gather(chunk, p):
            off = pl.multiple_of(chunk * _CHUNK, 8)
            pltpu.async_copy(
                table_hbm.at[idx_v.at[pl.ds(off, _CHUNK)]], bufs.at[p], gsem[p])

        def start_store(chunk, p):
            pltpu.async_copy(
                bufs.at[p], out_hbm.at[pl.ds(base + chunk * _CHUNK, _CHUNK)],
                ssem[p])

        def wait_gather(p):
            pltpu.make_async_copy(
                table_hbm.at[idx_v.at[pl.ds(0, _CHUNK)]], bufs.at[p],
                gsem[p]).wait()

        def wait_store(p):
            pltpu.make_async_copy(
                bufs.at[p], out_hbm.at[pl.ds(base, _CHUNK)], ssem[p]).wait()

        start_gather(0, 0)
        start_gather(1, 1)

        @pl.loop(0, n_main, step=3)
        def _chunk_loop(g):
            for p in (0, 1, 2):
                cur = g + p
                wait_gather(p)

                @pl.when(cur >= 1)
                def _():
                    wait_store((p + 2) % 3)

                start_gather(cur + 2, (p + 2) % 3)
                start_store(cur, p)

        for cur in (n_main, n_main + 1):
            p = cur % 3
            wait_gather(p)
            start_store(cur, p)
        for cur in (n_chunks - 3, n_chunks - 2, n_chunks - 1):
            wait_store(cur % 3)

    return body(ids_flat, table)


def kernel(input_ids, table):
    b, s = input_ids.shape
    d = table.shape[1]
    out = _gather_rows(input_ids.reshape(b * s), table)
    return out.reshape(b, s, d)
